# baseline (speedup 1.0000x reference)
import functools

import jax
import jax.numpy as jnp
from jax import lax
from jax.experimental import pallas as pl
from jax.experimental.pallas import tpu as pltpu
from jax.experimental.pallas import tpu_sc as plsc

_N = 4096 * 4096
_NC = 2
_NS = 16
_NW = _NC * _NS
_L = 16
_PER_W = _N // _NW
_SLAB_R = 8
_SLAB_C = 2048
_SLAB_ELEMS = _SLAB_R * _SLAB_C
_SLABS_PER_W = _PER_W // _SLAB_ELEMS
_SLABS_PER_ROWBAND = 4096 // _SLAB_C


def _make_hist_kernel(shift, nbins, maskc):
    hist_words = nbins * _L
    mesh = plsc.VectorSubcoreMesh(core_axis_name="c", subcore_axis_name="s")

    @functools.partial(
        pl.kernel,
        mesh=mesh,
        compiler_params=pltpu.CompilerParams(needs_layout_passes=False),
        out_type=jax.ShapeDtypeStruct((_NW, hist_words), jnp.int32),
        scratch_types=[
            pltpu.VMEM((_SLAB_R, _SLAB_C), jnp.float32),
            pltpu.VMEM((_SLAB_R, _SLAB_C), jnp.float32),
            pltpu.VMEM((hist_words,), jnp.int32),
            pltpu.VMEM((_L,), jnp.uint32),
            pltpu.SemaphoreType.DMA,
            pltpu.SemaphoreType.DMA,
        ],
    )
    def hist_kernel(scores_hbm, prefix_hbm, out_hbm, buf0, buf1, hist,
                    pref_v, sem0, sem1):
        wid = lax.axis_index("s") * _NC + lax.axis_index("c")

        zeros = jnp.zeros((_L,), jnp.int32)

        @plsc.parallel_loop(0, nbins, unroll=8)
        def _zero(i):
            hist[pl.ds(i * _L, _L)] = zeros

        pltpu.sync_copy(prefix_hbm, pref_v)
        pv = pref_v[...]
        lane = lax.iota(jnp.int32, _L)
        ones = jnp.ones((_L,), jnp.int32)

        def slab_src(l):
            g = wid * _SLABS_PER_W + l
            r0 = (g // _SLABS_PER_ROWBAND) * _SLAB_R
            c0 = (g % _SLABS_PER_ROWBAND) * _SLAB_C
            return scores_hbm.at[pl.ds(r0, _SLAB_R), pl.ds(c0, _SLAB_C)]

        def process(buf):
            @plsc.parallel_loop(0, _SLAB_ELEMS // _L, unroll=8)
            def _body(j):
                r = lax.shift_right_logical(j, 7)
                c = (j & (_SLAB_C // _L - 1)) * _L
                v = buf[r, pl.ds(c, _L)]
                bu = lax.bitcast_convert_type(v, jnp.uint32)
                if shift >= 4:
                    t = lax.shift_right_logical(bu, jnp.uint32(shift - 4))
                else:
                    t = lax.shift_left(bu, jnp.uint32(4 - shift))
                idx = lax.bitcast_convert_type(
                    t & jnp.uint32((nbins - 1) * _L), jnp.int32) | lane
                m = (bu & jnp.uint32(maskc)) == pv
                plsc.addupdate_scatter(hist, [idx], ones, mask=m)

        last = _SLABS_PER_W - 1
        pltpu.make_async_copy(slab_src(0), buf0, sem0).start()
        pltpu.make_async_copy(slab_src(1), buf1, sem1).start()

        def pair(p, _):
            l0 = 2 * p
            pltpu.make_async_copy(slab_src(l0), buf0, sem0).wait()
            process(buf0)
            pltpu.make_async_copy(
                slab_src(jnp.minimum(l0 + 2, last)), buf0, sem0).start()
            pltpu.make_async_copy(slab_src(l0 + 1), buf1, sem1).wait()
            process(buf1)
            pltpu.make_async_copy(
                slab_src(jnp.minimum(l0 + 3, last)), buf1, sem1).start()
            return 0

        lax.fori_loop(0, _SLABS_PER_W // 2, pair, 0)
        pltpu.make_async_copy(slab_src(last), buf0, sem0).wait()
        pltpu.make_async_copy(slab_src(last), buf1, sem1).wait()

        pltpu.sync_copy(hist, out_hbm.at[wid])

    return hist_kernel


_hist_a = _make_hist_kernel(20, 4096, 0)
_hist_b = _make_hist_kernel(8, 4096, 0xFFF00000)
_hist_c = _make_hist_kernel(0, 256, 0xFFFFFF00)


def _make_hist16_kernel(top):
    nbins = 65536
    mesh = plsc.VectorSubcoreMesh(core_axis_name="c", subcore_axis_name="s")

    @functools.partial(
        pl.kernel,
        mesh=mesh,
        compiler_params=pltpu.CompilerParams(needs_layout_passes=False),
        out_type=jax.ShapeDtypeStruct((_NW, nbins), jnp.int32),
        scratch_types=[
            pltpu.VMEM((_SLAB_R, _SLAB_C), jnp.float32),
            pltpu.VMEM((_SLAB_R, _SLAB_C), jnp.float32),
            pltpu.VMEM((nbins,), jnp.int32),
            pltpu.VMEM((_L,), jnp.uint32),
            pltpu.SemaphoreType.DMA,
            pltpu.SemaphoreType.DMA,
        ],
    )
    def hist_kernel(scores_hbm, prefix_hbm, out_hbm, buf0, buf1, hist,
                    pref_v, sem0, sem1):
        wid = lax.axis_index("s") * _NC + lax.axis_index("c")

        def slab_src(l):
            g = wid * _SLABS_PER_W + l
            r0 = (g // _SLABS_PER_ROWBAND) * _SLAB_R
            c0 = (g % _SLABS_PER_ROWBAND) * _SLAB_C
            return scores_hbm.at[pl.ds(r0, _SLAB_R), pl.ds(c0, _SLAB_C)]

        pltpu.make_async_copy(slab_src(0), buf0, sem0).start()
        pltpu.make_async_copy(slab_src(1), buf1, sem1).start()

        zeros = jnp.zeros((_L,), jnp.int32)

        @plsc.parallel_loop(0, nbins // _L, unroll=8)
        def _zero(i):
            hist[pl.ds(i * _L, _L)] = zeros

        pltpu.sync_copy(prefix_hbm, pref_v)
        pv = pref_v[...]
        ones = jnp.ones((_L,), jnp.int32)

        def process(buf):
            @plsc.parallel_loop(0, _SLAB_ELEMS // _L, unroll=16)
            def _body(j):
                r = lax.shift_right_logical(j, 7)
                c = (j & (_SLAB_C // _L - 1)) * _L
                v = buf[r, pl.ds(c, _L)]
                bu = lax.bitcast_convert_type(v, jnp.uint32)
                if top:
                    bucket = lax.shift_right_logical(bu, jnp.uint32(16))
                    m = (bu & jnp.uint32(0)) == pv
                else:
                    bucket = bu & jnp.uint32(0xFFFF)
                    m = (bu & jnp.uint32(0xFFFF0000)) == pv
                idx = lax.bitcast_convert_type(bucket, jnp.int32)
                plsc.addupdate_scatter(hist, [idx], ones, mask=m)

        last_slab = _SLABS_PER_W - 1

        def pair(p, _):
            l0 = 2 * p
            pltpu.make_async_copy(slab_src(l0), buf0, sem0).wait()
            process(buf0)
            pltpu.make_async_copy(
                slab_src(jnp.minimum(l0 + 2, last_slab)), buf0, sem0).start()
            pltpu.make_async_copy(slab_src(l0 + 1), buf1, sem1).wait()
            process(buf1)
            pltpu.make_async_copy(
                slab_src(jnp.minimum(l0 + 3, last_slab)), buf1, sem1).start()
            return 0

        lax.fori_loop(0, _SLABS_PER_W // 2, pair, 0)
        pltpu.make_async_copy(slab_src(last_slab), buf0, sem0).wait()
        pltpu.make_async_copy(slab_src(last_slab), buf1, sem1).wait()

        pltpu.sync_copy(hist, out_hbm.at[wid])

    return hist_kernel


_hist16_hi = _make_hist16_kernel(True)
_hist16_lo = _make_hist16_kernel(False)


def _reduce(hist_flat, nbins):
    return hist_flat.reshape(_NW, nbins, _L).sum(axis=(0, 2))


def _pick(h, rank):
    hb = h.reshape(-1, 16)
    blk_sums = hb.sum(axis=1)
    cb = jnp.cumsum(blk_sums)
    blk = jnp.argmax(cb >= rank)
    base = cb[blk] - blk_sums[blk]
    hw = lax.dynamic_slice_in_dim(hb, blk, 1, 0)[0]
    cw = jnp.cumsum(hw) + base
    off = jnp.argmax(cw >= rank)
    within = rank - (cw[off] - hw[off])
    return blk * 16 + off, within


def _apply_body(thr_ref, w_ref, s_ref, o_ref):
    thr = thr_ref[0, 0]
    o_ref[...] = jnp.where(s_ref[...] < thr, jnp.float32(0.0), w_ref[...])


_apply = pl.pallas_call(
    _apply_body,
    grid=(8,),
    in_specs=[
        pl.BlockSpec(memory_space=pltpu.SMEM),
        pl.BlockSpec((512, 4096), lambda i: (i, 0)),
        pl.BlockSpec((512, 4096), lambda i: (i, 0)),
    ],
    out_specs=pl.BlockSpec((512, 4096), lambda i: (i, 0)),
    out_shape=jax.ShapeDtypeStruct((4096, 4096), jnp.float32),
)


def kernel(weight, scores):
    n = scores.size
    k = jnp.int32(int(1 + round(0.9 * (n - 1))))

    h_raw = _hist16_hi(scores, jnp.zeros((_L,), jnp.uint32)).sum(axis=0)
    h_key = jnp.concatenate([h_raw[32768:][::-1], h_raw[:32768]])
    b1, r1 = _pick(h_key, k)
    neg = b1 < 32768
    r1raw = jnp.where(neg, 65535 - b1, b1 - 32768)
    p1 = r1raw.astype(jnp.uint32) << 16

    h2 = _hist16_lo(scores, jnp.broadcast_to(p1, (_L,))).sum(axis=0)
    b2, _ = _pick(jnp.where(neg, h2[::-1], h2), r1)
    lowraw = jnp.where(neg, 65535 - b2, b2)

    bits = p1 | lowraw.astype(jnp.uint32)
    thr = lax.bitcast_convert_type(bits, jnp.float32)

    return _apply(thr.reshape(1, 1), weight, scores)


# speedup vs baseline: 1.0018x; 1.0018x over previous
import functools

import jax
import jax.numpy as jnp
from jax import lax
from jax.experimental import pallas as pl
from jax.experimental.pallas import tpu as pltpu
from jax.experimental.pallas import tpu_sc as plsc

_N = 4096 * 4096
_NC = 2
_NS = 16
_NW = _NC * _NS
_L = 16
_PER_W = _N // _NW
_SLAB_R = 8
_SLAB_C = 2048
_SLAB_ELEMS = _SLAB_R * _SLAB_C
_SLABS_PER_W = _PER_W // _SLAB_ELEMS
_SLABS_PER_ROWBAND = 4096 // _SLAB_C


def _make_hist_kernel(shift, nbins, maskc):
    hist_words = nbins * _L
    mesh = plsc.VectorSubcoreMesh(core_axis_name="c", subcore_axis_name="s")

    @functools.partial(
        pl.kernel,
        mesh=mesh,
        compiler_params=pltpu.CompilerParams(needs_layout_passes=False),
        out_type=jax.ShapeDtypeStruct((_NW, hist_words), jnp.int32),
        scratch_types=[
            pltpu.VMEM((_SLAB_R, _SLAB_C), jnp.float32),
            pltpu.VMEM((_SLAB_R, _SLAB_C), jnp.float32),
            pltpu.VMEM((hist_words,), jnp.int32),
            pltpu.VMEM((_L,), jnp.uint32),
            pltpu.SemaphoreType.DMA,
            pltpu.SemaphoreType.DMA,
        ],
    )
    def hist_kernel(scores_hbm, prefix_hbm, out_hbm, buf0, buf1, hist,
                    pref_v, sem0, sem1):
        wid = lax.axis_index("s") * _NC + lax.axis_index("c")

        zeros = jnp.zeros((_L,), jnp.int32)

        @plsc.parallel_loop(0, nbins, unroll=8)
        def _zero(i):
            hist[pl.ds(i * _L, _L)] = zeros

        pltpu.sync_copy(prefix_hbm, pref_v)
        pv = pref_v[...]
        lane = lax.iota(jnp.int32, _L)
        ones = jnp.ones((_L,), jnp.int32)

        def slab_src(l):
            g = wid * _SLABS_PER_W + l
            r0 = (g // _SLABS_PER_ROWBAND) * _SLAB_R
            c0 = (g % _SLABS_PER_ROWBAND) * _SLAB_C
            return scores_hbm.at[pl.ds(r0, _SLAB_R), pl.ds(c0, _SLAB_C)]

        def process(buf):
            @plsc.parallel_loop(0, _SLAB_ELEMS // _L, unroll=8)
            def _body(j):
                r = lax.shift_right_logical(j, 7)
                c = (j & (_SLAB_C // _L - 1)) * _L
                v = buf[r, pl.ds(c, _L)]
                bu = lax.bitcast_convert_type(v, jnp.uint32)
                if shift >= 4:
                    t = lax.shift_right_logical(bu, jnp.uint32(shift - 4))
                else:
                    t = lax.shift_left(bu, jnp.uint32(4 - shift))
                idx = lax.bitcast_convert_type(
                    t & jnp.uint32((nbins - 1) * _L), jnp.int32) | lane
                m = (bu & jnp.uint32(maskc)) == pv
                plsc.addupdate_scatter(hist, [idx], ones, mask=m)

        last = _SLABS_PER_W - 1
        pltpu.make_async_copy(slab_src(0), buf0, sem0).start()
        pltpu.make_async_copy(slab_src(1), buf1, sem1).start()

        def pair(p, _):
            l0 = 2 * p
            pltpu.make_async_copy(slab_src(l0), buf0, sem0).wait()
            process(buf0)
            pltpu.make_async_copy(
                slab_src(jnp.minimum(l0 + 2, last)), buf0, sem0).start()
            pltpu.make_async_copy(slab_src(l0 + 1), buf1, sem1).wait()
            process(buf1)
            pltpu.make_async_copy(
                slab_src(jnp.minimum(l0 + 3, last)), buf1, sem1).start()
            return 0

        lax.fori_loop(0, _SLABS_PER_W // 2, pair, 0)
        pltpu.make_async_copy(slab_src(last), buf0, sem0).wait()
        pltpu.make_async_copy(slab_src(last), buf1, sem1).wait()

        pltpu.sync_copy(hist, out_hbm.at[wid])

    return hist_kernel


_hist_a = _make_hist_kernel(20, 4096, 0)
_hist_b = _make_hist_kernel(8, 4096, 0xFFF00000)
_hist_c = _make_hist_kernel(0, 256, 0xFFFFFF00)


def _make_hist16_kernel(top):
    nbins = 65536
    mesh = plsc.VectorSubcoreMesh(core_axis_name="c", subcore_axis_name="s")

    @functools.partial(
        pl.kernel,
        mesh=mesh,
        compiler_params=pltpu.CompilerParams(needs_layout_passes=False),
        out_type=jax.ShapeDtypeStruct((_NW, nbins), jnp.int32),
        scratch_types=[
            pltpu.VMEM((_SLAB_R, _SLAB_C), jnp.float32),
            pltpu.VMEM((_SLAB_R, _SLAB_C), jnp.float32),
            pltpu.VMEM((nbins,), jnp.int32),
            pltpu.VMEM((_L,), jnp.uint32),
            pltpu.SemaphoreType.DMA,
            pltpu.SemaphoreType.DMA,
        ],
    )
    def hist_kernel(scores_hbm, prefix_hbm, out_hbm, buf0, buf1, hist,
                    pref_v, sem0, sem1):
        wid = lax.axis_index("s") * _NC + lax.axis_index("c")

        def slab_src(l):
            g = wid * _SLABS_PER_W + l
            r0 = (g // _SLABS_PER_ROWBAND) * _SLAB_R
            c0 = (g % _SLABS_PER_ROWBAND) * _SLAB_C
            return scores_hbm.at[pl.ds(r0, _SLAB_R), pl.ds(c0, _SLAB_C)]

        pltpu.make_async_copy(slab_src(0), buf0, sem0).start()
        pltpu.make_async_copy(slab_src(1), buf1, sem1).start()

        zeros = jnp.zeros((_L,), jnp.int32)

        @plsc.parallel_loop(0, nbins // _L, unroll=8)
        def _zero(i):
            hist[pl.ds(i * _L, _L)] = zeros

        pltpu.sync_copy(prefix_hbm, pref_v)
        pv = pref_v[...]
        ones = jnp.ones((_L,), jnp.int32)

        def process(buf):
            @plsc.parallel_loop(0, _SLAB_ELEMS // _L, unroll=16)
            def _body(j):
                r = lax.shift_right_logical(j, 7)
                c = (j & (_SLAB_C // _L - 1)) * _L
                v = buf[r, pl.ds(c, _L)]
                bu = lax.bitcast_convert_type(v, jnp.uint32)
                if top:
                    bucket = lax.shift_right_logical(bu, jnp.uint32(16))
                    m = (bu & jnp.uint32(0)) == pv
                else:
                    bucket = bu & jnp.uint32(0xFFFF)
                    m = (bu & jnp.uint32(0xFFFF0000)) == pv
                idx = lax.bitcast_convert_type(bucket, jnp.int32)
                plsc.addupdate_scatter(hist, [idx], ones, mask=m)

        last_slab = _SLABS_PER_W - 1

        def pair(p, _):
            l0 = 2 * p
            pltpu.make_async_copy(slab_src(l0), buf0, sem0).wait()
            process(buf0)
            pltpu.make_async_copy(
                slab_src(jnp.minimum(l0 + 2, last_slab)), buf0, sem0).start()
            pltpu.make_async_copy(slab_src(l0 + 1), buf1, sem1).wait()
            process(buf1)
            pltpu.make_async_copy(
                slab_src(jnp.minimum(l0 + 3, last_slab)), buf1, sem1).start()
            return 0

        lax.fori_loop(0, _SLABS_PER_W // 2, pair, 0)
        pltpu.make_async_copy(slab_src(last_slab), buf0, sem0).wait()
        pltpu.make_async_copy(slab_src(last_slab), buf1, sem1).wait()

        pltpu.sync_copy(hist, out_hbm.at[wid])

    return hist_kernel


_hist16_hi = _make_hist16_kernel(True)
_hist16_lo = _make_hist16_kernel(False)


def _reduce(hist_flat, nbins):
    return hist_flat.reshape(_NW, nbins, _L).sum(axis=(0, 2))


def _pick(h, rank):
    hb = h.reshape(-1, 16)
    blk_sums = hb.sum(axis=1)
    cb = jnp.cumsum(blk_sums)
    blk = jnp.argmax(cb >= rank)
    base = cb[blk] - blk_sums[blk]
    hw = lax.dynamic_slice_in_dim(hb, blk, 1, 0)[0]
    cw = jnp.cumsum(hw) + base
    off = jnp.argmax(cw >= rank)
    within = rank - (cw[off] - hw[off])
    return blk * 16 + off, within


def _apply_body(thr_ref, w_ref, s_ref, o_ref):
    thr = thr_ref[0, 0]
    o_ref[...] = jnp.where(s_ref[...] < thr, jnp.float32(0.0), w_ref[...])


_apply = pl.pallas_call(
    _apply_body,
    grid=(16,),
    in_specs=[
        pl.BlockSpec(memory_space=pltpu.SMEM),
        pl.BlockSpec((256, 4096), lambda i: (i, 0)),
        pl.BlockSpec((256, 4096), lambda i: (i, 0)),
    ],
    out_specs=pl.BlockSpec((256, 4096), lambda i: (i, 0)),
    out_shape=jax.ShapeDtypeStruct((4096, 4096), jnp.float32),
)


def kernel(weight, scores):
    n = scores.size
    k = jnp.int32(int(1 + round(0.9 * (n - 1))))

    h_raw = _hist16_hi(scores, jnp.zeros((_L,), jnp.uint32)).sum(axis=0)
    h_key = jnp.concatenate([h_raw[32768:][::-1], h_raw[:32768]])
    b1, r1 = _pick(h_key, k)
    neg = b1 < 32768
    r1raw = jnp.where(neg, 65535 - b1, b1 - 32768)
    p1 = r1raw.astype(jnp.uint32) << 16

    h2 = _hist16_lo(scores, jnp.broadcast_to(p1, (_L,))).sum(axis=0)
    b2, _ = _pick(jnp.where(neg, h2[::-1], h2), r1)
    lowraw = jnp.where(neg, 65535 - b2, b2)

    bits = p1 | lowraw.astype(jnp.uint32)
    thr = lax.bitcast_convert_type(bits, jnp.float32)

    return _apply(thr.reshape(1, 1), weight, scores)


# speedup vs baseline: 1.0543x; 1.0524x over previous
import functools

import jax
import jax.numpy as jnp
from jax import lax
from jax.experimental import pallas as pl
from jax.experimental.pallas import tpu as pltpu
from jax.experimental.pallas import tpu_sc as plsc

_N = 4096 * 4096
_NC = 2
_NS = 16
_NW = _NC * _NS
_L = 16
_PER_W = _N // _NW
_SLAB_R = 8
_SLAB_C = 2048
_SLAB_ELEMS = _SLAB_R * _SLAB_C
_SLABS_PER_W = _PER_W // _SLAB_ELEMS
_SLABS_PER_ROWBAND = 4096 // _SLAB_C


def _make_hist_kernel(shift, nbins, maskc):
    hist_words = nbins * _L
    mesh = plsc.VectorSubcoreMesh(core_axis_name="c", subcore_axis_name="s")

    @functools.partial(
        pl.kernel,
        mesh=mesh,
        compiler_params=pltpu.CompilerParams(needs_layout_passes=False),
        out_type=jax.ShapeDtypeStruct((_NW, hist_words), jnp.int32),
        scratch_types=[
            pltpu.VMEM((_SLAB_R, _SLAB_C), jnp.float32),
            pltpu.VMEM((_SLAB_R, _SLAB_C), jnp.float32),
            pltpu.VMEM((hist_words,), jnp.int32),
            pltpu.VMEM((_L,), jnp.uint32),
            pltpu.SemaphoreType.DMA,
            pltpu.SemaphoreType.DMA,
        ],
    )
    def hist_kernel(scores_hbm, prefix_hbm, out_hbm, buf0, buf1, hist,
                    pref_v, sem0, sem1):
        wid = lax.axis_index("s") * _NC + lax.axis_index("c")

        zeros = jnp.zeros((_L,), jnp.int32)

        @plsc.parallel_loop(0, nbins, unroll=8)
        def _zero(i):
            hist[pl.ds(i * _L, _L)] = zeros

        pltpu.sync_copy(prefix_hbm, pref_v)
        pv = pref_v[...]
        lane = lax.iota(jnp.int32, _L)
        ones = jnp.ones((_L,), jnp.int32)

        def slab_src(l):
            g = wid * _SLABS_PER_W + l
            r0 = (g // _SLABS_PER_ROWBAND) * _SLAB_R
            c0 = (g % _SLABS_PER_ROWBAND) * _SLAB_C
            return scores_hbm.at[pl.ds(r0, _SLAB_R), pl.ds(c0, _SLAB_C)]

        def process(buf):
            @plsc.parallel_loop(0, _SLAB_ELEMS // _L, unroll=8)
            def _body(j):
                r = lax.shift_right_logical(j, 7)
                c = (j & (_SLAB_C // _L - 1)) * _L
                v = buf[r, pl.ds(c, _L)]
                bu = lax.bitcast_convert_type(v, jnp.uint32)
                if shift >= 4:
                    t = lax.shift_right_logical(bu, jnp.uint32(shift - 4))
                else:
                    t = lax.shift_left(bu, jnp.uint32(4 - shift))
                idx = lax.bitcast_convert_type(
                    t & jnp.uint32((nbins - 1) * _L), jnp.int32) | lane
                m = (bu & jnp.uint32(maskc)) == pv
                plsc.addupdate_scatter(hist, [idx], ones, mask=m)

        last = _SLABS_PER_W - 1
        pltpu.make_async_copy(slab_src(0), buf0, sem0).start()
        pltpu.make_async_copy(slab_src(1), buf1, sem1).start()

        def pair(p, _):
            l0 = 2 * p
            pltpu.make_async_copy(slab_src(l0), buf0, sem0).wait()
            process(buf0)
            pltpu.make_async_copy(
                slab_src(jnp.minimum(l0 + 2, last)), buf0, sem0).start()
            pltpu.make_async_copy(slab_src(l0 + 1), buf1, sem1).wait()
            process(buf1)
            pltpu.make_async_copy(
                slab_src(jnp.minimum(l0 + 3, last)), buf1, sem1).start()
            return 0

        lax.fori_loop(0, _SLABS_PER_W // 2, pair, 0)
        pltpu.make_async_copy(slab_src(last), buf0, sem0).wait()
        pltpu.make_async_copy(slab_src(last), buf1, sem1).wait()

        pltpu.sync_copy(hist, out_hbm.at[wid])

    return hist_kernel


_hist_a = _make_hist_kernel(20, 4096, 0)
_hist_b = _make_hist_kernel(8, 4096, 0xFFF00000)
_hist_c = _make_hist_kernel(0, 256, 0xFFFFFF00)


def _make_hist16_kernel(top):
    nbins = 65536
    mesh = plsc.VectorSubcoreMesh(core_axis_name="c", subcore_axis_name="s")

    @functools.partial(
        pl.kernel,
        mesh=mesh,
        compiler_params=pltpu.CompilerParams(needs_layout_passes=False),
        out_type=jax.ShapeDtypeStruct((_NW, nbins), jnp.int32),
        scratch_types=[
            pltpu.VMEM((_SLAB_R, _SLAB_C), jnp.float32),
            pltpu.VMEM((_SLAB_R, _SLAB_C), jnp.float32),
            pltpu.VMEM((nbins,), jnp.int32),
            pltpu.VMEM((_L,), jnp.uint32),
            pltpu.SemaphoreType.DMA,
            pltpu.SemaphoreType.DMA,
        ],
    )
    def hist_kernel(scores_hbm, prefix_hbm, out_hbm, buf0, buf1, hist,
                    pref_v, sem0, sem1):
        wid = lax.axis_index("s") * _NC + lax.axis_index("c")

        def slab_src(l):
            g = wid * _SLABS_PER_W + l
            r0 = (g // _SLABS_PER_ROWBAND) * _SLAB_R
            c0 = (g % _SLABS_PER_ROWBAND) * _SLAB_C
            return scores_hbm.at[pl.ds(r0, _SLAB_R), pl.ds(c0, _SLAB_C)]

        pltpu.make_async_copy(slab_src(0), buf0, sem0).start()
        pltpu.make_async_copy(slab_src(1), buf1, sem1).start()

        zeros = jnp.zeros((_L,), jnp.int32)

        @plsc.parallel_loop(0, nbins // _L, unroll=8)
        def _zero(i):
            hist[pl.ds(i * _L, _L)] = zeros

        pltpu.sync_copy(prefix_hbm, pref_v)
        pv = pref_v[...]
        ones = jnp.ones((_L,), jnp.int32)

        def process(buf):
            @plsc.parallel_loop(0, _SLAB_ELEMS // _L, unroll=16)
            def _body(j):
                r = lax.shift_right_logical(j, 7)
                c = (j & (_SLAB_C // _L - 1)) * _L
                v = buf[r, pl.ds(c, _L)]
                bu = lax.bitcast_convert_type(v, jnp.uint32)
                if top:
                    bucket = lax.shift_right_logical(bu, jnp.uint32(16))
                    m = (bu & jnp.uint32(0)) == pv
                else:
                    bucket = bu & jnp.uint32(0xFFFF)
                    m = (bu & jnp.uint32(0xFFFF0000)) == pv
                idx = lax.bitcast_convert_type(bucket, jnp.int32)
                plsc.addupdate_scatter(hist, [idx], ones, mask=m)

        last_slab = _SLABS_PER_W - 1

        def pair(p, _):
            l0 = 2 * p
            pltpu.make_async_copy(slab_src(l0), buf0, sem0).wait()
            process(buf0)
            pltpu.make_async_copy(
                slab_src(jnp.minimum(l0 + 2, last_slab)), buf0, sem0).start()
            pltpu.make_async_copy(slab_src(l0 + 1), buf1, sem1).wait()
            process(buf1)
            pltpu.make_async_copy(
                slab_src(jnp.minimum(l0 + 3, last_slab)), buf1, sem1).start()
            return 0

        lax.fori_loop(0, _SLABS_PER_W // 2, pair, 0)
        pltpu.make_async_copy(slab_src(last_slab), buf0, sem0).wait()
        pltpu.make_async_copy(slab_src(last_slab), buf1, sem1).wait()

        pltpu.sync_copy(hist, out_hbm.at[wid])

    return hist_kernel


_hist16_hi = _make_hist16_kernel(True)
_hist16_lo = _make_hist16_kernel(False)


def _reduce(hist_flat, nbins):
    return hist_flat.reshape(_NW, nbins, _L).sum(axis=(0, 2))


def _pick(h, rank):
    c = jnp.cumsum(h)
    b = jnp.argmax(c >= rank)
    within = rank - (c[b] - h[b])
    return b, within


def _apply_body(thr_ref, w_ref, s_ref, o_ref):
    thr = thr_ref[0, 0]
    o_ref[...] = jnp.where(s_ref[...] < thr, jnp.float32(0.0), w_ref[...])


_apply = pl.pallas_call(
    _apply_body,
    grid=(16,),
    in_specs=[
        pl.BlockSpec(memory_space=pltpu.SMEM),
        pl.BlockSpec((256, 4096), lambda i: (i, 0)),
        pl.BlockSpec((256, 4096), lambda i: (i, 0)),
    ],
    out_specs=pl.BlockSpec((256, 4096), lambda i: (i, 0)),
    out_shape=jax.ShapeDtypeStruct((4096, 4096), jnp.float32),
)


def kernel(weight, scores):
    n = scores.size
    k = jnp.int32(int(1 + round(0.9 * (n - 1))))

    h_raw = _hist16_hi(scores, jnp.zeros((_L,), jnp.uint32)).sum(axis=0)
    h_key = jnp.concatenate([h_raw[32768:][::-1], h_raw[:32768]])
    b1, r1 = _pick(h_key, k)
    neg = b1 < 32768
    r1raw = jnp.where(neg, 65535 - b1, b1 - 32768)
    p1 = r1raw.astype(jnp.uint32) << 16

    h2 = _hist16_lo(scores, jnp.broadcast_to(p1, (_L,))).sum(axis=0)
    b2, _ = _pick(jnp.where(neg, h2[::-1], h2), r1)
    lowraw = jnp.where(neg, 65535 - b2, b2)

    bits = p1 | lowraw.astype(jnp.uint32)
    thr = lax.bitcast_convert_type(bits, jnp.float32)

    return _apply(thr.reshape(1, 1), weight, scores)


# speedup vs baseline: 1.0552x; 1.0009x over previous
import functools

import jax
import jax.numpy as jnp
from jax import lax
from jax.experimental import pallas as pl
from jax.experimental.pallas import tpu as pltpu
from jax.experimental.pallas import tpu_sc as plsc

_N = 4096 * 4096
_NC = 2
_NS = 16
_NW = _NC * _NS
_L = 16
_PER_W = _N // _NW
_SLAB_R = 8
_SLAB_C = 2048
_SLAB_ELEMS = _SLAB_R * _SLAB_C
_SLABS_PER_W = _PER_W // _SLAB_ELEMS
_SLABS_PER_ROWBAND = 4096 // _SLAB_C


def _make_hist16_kernel(top):
    nbins = 65536
    mesh = plsc.VectorSubcoreMesh(core_axis_name="c", subcore_axis_name="s")

    @functools.partial(
        pl.kernel,
        mesh=mesh,
        compiler_params=pltpu.CompilerParams(needs_layout_passes=False),
        out_type=jax.ShapeDtypeStruct((_NW, nbins), jnp.int32),
        scratch_types=[
            pltpu.VMEM((_SLAB_R, _SLAB_C), jnp.float32),
            pltpu.VMEM((_SLAB_R, _SLAB_C), jnp.float32),
            pltpu.VMEM((nbins,), jnp.int32),
            pltpu.VMEM((_L,), jnp.uint32),
            pltpu.SemaphoreType.DMA,
            pltpu.SemaphoreType.DMA,
        ],
    )
    def hist_kernel(scores_hbm, prefix_hbm, out_hbm, buf0, buf1, hist,
                    pref_v, sem0, sem1):
        wid = lax.axis_index("s") * _NC + lax.axis_index("c")

        def slab_src(l):
            g = wid * _SLABS_PER_W + l
            r0 = (g // _SLABS_PER_ROWBAND) * _SLAB_R
            c0 = (g % _SLABS_PER_ROWBAND) * _SLAB_C
            return scores_hbm.at[pl.ds(r0, _SLAB_R), pl.ds(c0, _SLAB_C)]

        pltpu.make_async_copy(slab_src(0), buf0, sem0).start()
        pltpu.make_async_copy(slab_src(1), buf1, sem1).start()

        zeros = jnp.zeros((_L,), jnp.int32)

        @plsc.parallel_loop(0, nbins // _L, unroll=8)
        def _zero(i):
            hist[pl.ds(i * _L, _L)] = zeros

        pltpu.sync_copy(prefix_hbm, pref_v)
        pv = pref_v[...]
        ones = jnp.ones((_L,), jnp.int32)

        def process(buf):
            @plsc.parallel_loop(0, _SLAB_ELEMS // _L, unroll=16)
            def _body(j):
                r = lax.shift_right_logical(j, 7)
                c = (j & (_SLAB_C // _L - 1)) * _L
                v = buf[r, pl.ds(c, _L)]
                bu = lax.bitcast_convert_type(v, jnp.uint32)
                if top:
                    bucket = lax.shift_right_logical(bu, jnp.uint32(16))
                    m = (bu & jnp.uint32(0)) == pv
                else:
                    bucket = bu & jnp.uint32(0xFFFF)
                    m = (bu & jnp.uint32(0xFFFF0000)) == pv
                idx = lax.bitcast_convert_type(bucket, jnp.int32)
                plsc.addupdate_scatter(hist, [idx], ones, mask=m)

        last_slab = _SLABS_PER_W - 1

        def pair(p, _):
            l0 = 2 * p
            pltpu.make_async_copy(slab_src(l0), buf0, sem0).wait()
            process(buf0)
            pltpu.make_async_copy(
                slab_src(jnp.minimum(l0 + 2, last_slab)), buf0, sem0).start()
            pltpu.make_async_copy(slab_src(l0 + 1), buf1, sem1).wait()
            process(buf1)
            pltpu.make_async_copy(
                slab_src(jnp.minimum(l0 + 3, last_slab)), buf1, sem1).start()
            return 0

        lax.fori_loop(0, _SLABS_PER_W // 2, pair, 0)
        pltpu.make_async_copy(slab_src(last_slab), buf0, sem0).wait()
        pltpu.make_async_copy(slab_src(last_slab), buf1, sem1).wait()

        pltpu.sync_copy(hist, out_hbm.at[wid])

    return hist_kernel


_hist16_hi = _make_hist16_kernel(True)
_hist16_lo = _make_hist16_kernel(False)


def _pick(h, rank):
    c = jnp.cumsum(h)
    b = jnp.argmax(c >= rank)
    within = rank - (c[b] - h[b])
    return b, within


def _apply_body(thr_ref, w_ref, s_ref, o_ref):
    thr = thr_ref[0, 0]
    o_ref[...] = jnp.where(s_ref[...] < thr, jnp.float32(0.0), w_ref[...])


_apply = pl.pallas_call(
    _apply_body,
    grid=(16,),
    in_specs=[
        pl.BlockSpec(memory_space=pltpu.SMEM),
        pl.BlockSpec((256, 4096), lambda i: (i, 0)),
        pl.BlockSpec((256, 4096), lambda i: (i, 0)),
    ],
    out_specs=pl.BlockSpec((256, 4096), lambda i: (i, 0)),
    out_shape=jax.ShapeDtypeStruct((4096, 4096), jnp.float32),
)


def kernel(weight, scores):
    n = scores.size
    k = jnp.int32(int(1 + round(0.9 * (n - 1))))

    h_raw = _hist16_hi(scores, jnp.zeros((_L,), jnp.uint32)).sum(axis=0)
    h_key = jnp.concatenate([h_raw[32768:][::-1], h_raw[:32768]])
    b1, r1 = _pick(h_key, k)
    neg = b1 < 32768
    r1raw = jnp.where(neg, 65535 - b1, b1 - 32768)
    p1 = r1raw.astype(jnp.uint32) << 16

    h2 = _hist16_lo(scores, jnp.broadcast_to(p1, (_L,))).sum(axis=0)
    b2, _ = _pick(jnp.where(neg, h2[::-1], h2), r1)
    lowraw = jnp.where(neg, 65535 - b2, b2)

    bits = p1 | lowraw.astype(jnp.uint32)
    thr = lax.bitcast_convert_type(bits, jnp.float32)

    return _apply(thr.reshape(1, 1), weight, scores)
